# Initial kernel scaffold; baseline (speedup 1.0000x reference)
import jax
import jax.numpy as jnp
from jax.experimental import pallas as pl


def kernel(feat, turn, control, intersect, map_ctrs, actors, actor_ctrs, params):
    raise NotImplementedError("write your pallas kernel here")


# speedup vs baseline: 7.0530x; 7.0530x over previous
import functools

import jax
import jax.numpy as jnp
from jax import lax
from jax.experimental import pallas as pl
from jax.experimental.pallas import tpu as pltpu

N_MAP = 10000
N_ACT = 1600
D = 128
NW = 32
WROWS = 320
MPAD = NW * WROWS
SEG = 16384
E_CAP = NW * SEG
BLK = 1024
PB = SEG // BLK
RB = 2000

_INTERPRET = False


def _gn(y, g, b):
    m = jnp.mean(y, axis=1, keepdims=True)
    v = jnp.mean((y - m) ** 2, axis=1, keepdims=True)
    return (y - m) / jnp.sqrt(v + 1e-5) * g + b


def _relu(x):
    return jnp.maximum(x, 0.0)


def _k0_body(feat_ref, meta_ref, wft_ref, wmt_ref, g_ref, b_ref, o_ref):
    y = jnp.dot(feat_ref[...], wft_ref[...], preferred_element_type=jnp.float32)
    meta = meta_ref[...]
    wmt = wmt_ref[...]
    for i in range(4):
        y = y + meta[:, i:i + 1] * wmt[i:i + 1, :]
    o_ref[...] = _relu(_gn(y, g_ref[...], b_ref[...]))


def _k0(feat, meta, wft, wmt, g, b):
    grid = (N_MAP // RB,)
    return pl.pallas_call(
        _k0_body,
        grid=grid,
        in_specs=[
            pl.BlockSpec((RB, D), lambda i: (i, 0)),
            pl.BlockSpec((RB, 4), lambda i: (i, 0)),
            pl.BlockSpec((D, D), lambda i: (0, 0)),
            pl.BlockSpec((4, D), lambda i: (0, 0)),
            pl.BlockSpec((1, D), lambda i: (0, 0)),
            pl.BlockSpec((1, D), lambda i: (0, 0)),
        ],
        out_specs=pl.BlockSpec((RB, D), lambda i: (i, 0)),
        out_shape=jax.ShapeDtypeStruct((N_MAP, D), jnp.float32),
        interpret=_INTERPRET,
    )(feat, meta, wft, wmt, g, b)


def _k1_body(x_ref, act_ref, agtT_ref, qT_ref, qg_ref, qb_ref, wqT_ref,
             wcT_ref, a0_ref, qc_ref, cc_ref):
    a0_ref[...] = jnp.dot(x_ref[...], agtT_ref[...],
                          preferred_element_type=jnp.float32)
    q = _relu(_gn(jnp.dot(x_ref[...], qT_ref[...],
                          preferred_element_type=jnp.float32),
                  qg_ref[...], qb_ref[...]))
    qc_ref[...] = jnp.dot(q, wqT_ref[...], preferred_element_type=jnp.float32)

    @pl.when(pl.program_id(0) == 0)
    def _():
        cc_ref[...] = jnp.dot(act_ref[...], wcT_ref[...],
                              preferred_element_type=jnp.float32)


def _k1(x, actors, agtT, qT, qg, qb, wqT, wcT):
    grid = (N_MAP // RB,)
    return pl.pallas_call(
        _k1_body,
        grid=grid,
        in_specs=[
            pl.BlockSpec((RB, D), lambda i: (i, 0)),
            pl.BlockSpec((N_ACT, D), lambda i: (0, 0)),
            pl.BlockSpec((D, D), lambda i: (0, 0)),
            pl.BlockSpec((D, D), lambda i: (0, 0)),
            pl.BlockSpec((1, D), lambda i: (0, 0)),
            pl.BlockSpec((1, D), lambda i: (0, 0)),
            pl.BlockSpec((D, D), lambda i: (0, 0)),
            pl.BlockSpec((D, D), lambda i: (0, 0)),
        ],
        out_specs=[
            pl.BlockSpec((RB, D), lambda i: (i, 0)),
            pl.BlockSpec((RB, D), lambda i: (i, 0)),
            pl.BlockSpec((N_ACT, D), lambda i: (0, 0)),
        ],
        out_shape=[
            jax.ShapeDtypeStruct((N_MAP, D), jnp.float32),
            jax.ShapeDtypeStruct((N_MAP, D), jnp.float32),
            jax.ShapeDtypeStruct((N_ACT, D), jnp.float32),
        ],
        interpret=_INTERPRET,
    )(x, actors, agtT, qT, qg, qb, wqT, wcT)


def _k4_body(cnt_ref, relx_ref, rely_ref, qg_ref, cg_ref,
             d0x_ref, d0y_ref, d0b_ref, w1T_ref, d1g_ref, d1b_ref,
             wdT_ref, c0g_ref, c0b_ref, c1T_ref, o_ref):
    pid = pl.program_id(0)
    s = pid // PB
    base = (pid % PB) * BLK
    cnt = cnt_ref[s, 0]

    @pl.when(base < cnt)
    def _():
        d0 = _relu(relx_ref[...] * d0x_ref[...] + rely_ref[...] * d0y_ref[...]
                   + d0b_ref[...])
        d1 = _relu(_gn(jnp.dot(d0, w1T_ref[...],
                               preferred_element_type=jnp.float32),
                       d1g_ref[...], d1b_ref[...]))
        e = (jnp.dot(d1, wdT_ref[...], preferred_element_type=jnp.float32)
             + qg_ref[...] + cg_ref[...])
        c1 = _relu(_gn(e, c0g_ref[...], c0b_ref[...]))
        c = jnp.dot(c1, c1T_ref[...], preferred_element_type=jnp.float32)
        row = base + lax.broadcasted_iota(jnp.int32, (BLK, 1), 0)
        o_ref[...] = jnp.where(row < cnt, c, 0.0)


def _k4(counts, relx, rely, qg, cg, p):
    grid = (NW * PB,)
    wspec = pl.BlockSpec((D, D), lambda i: (0, 0))
    vspec = pl.BlockSpec((1, D), lambda i: (0, 0))
    espec = pl.BlockSpec((BLK, D), lambda i: (i, 0))
    sspec = pl.BlockSpec((BLK, 1), lambda i: (i, 0))
    return pl.pallas_call(
        _k4_body,
        grid=grid,
        in_specs=[
            pl.BlockSpec(memory_space=pltpu.SMEM),
            sspec, sspec, espec, espec,
            vspec, vspec, vspec, wspec, vspec, vspec,
            wspec, vspec, vspec, wspec,
        ],
        out_specs=espec,
        out_shape=jax.ShapeDtypeStruct((E_CAP, D), jnp.float32),
        interpret=_INTERPRET,
    )(counts, relx, rely, qg, cg,
      p['d0x'], p['d0y'], p['d0b'], p['w1T'], p['d1g'], p['d1b'],
      p['wdT'], p['c0g'], p['c0b'], p['c1T'])


def _k6_body(a0_ref, p0_ref, p1_ref, res_ref, linT_ref,
             ng_ref, nb_ref, lg_ref, lb_ref, o_ref):
    a = a0_ref[...] + p0_ref[...] + p1_ref[...]
    h = _relu(_gn(a, ng_ref[...], nb_ref[...]))
    h2 = _gn(jnp.dot(h, linT_ref[...], preferred_element_type=jnp.float32),
             lg_ref[...], lb_ref[...])
    o_ref[...] = _relu(h2 + res_ref[...])


def _k6(a0, p0, p1, res, linT, ng, nb, lg, lb):
    grid = (N_MAP // RB,)
    rspec = pl.BlockSpec((RB, D), lambda i: (i, 0))
    wspec = pl.BlockSpec((D, D), lambda i: (0, 0))
    vspec = pl.BlockSpec((1, D), lambda i: (0, 0))
    return pl.pallas_call(
        _k6_body,
        grid=grid,
        in_specs=[rspec, rspec, rspec, rspec, wspec,
                  vspec, vspec, vspec, vspec],
        out_specs=rspec,
        out_shape=jax.ShapeDtypeStruct((N_MAP, D), jnp.float32),
        interpret=_INTERPRET,
    )(a0, p0, p1, res, linT, ng, nb, lg, lb)


def _edge_build_jnp(map_ctrs, actor_ctrs):
    diff = map_ctrs[:, None, :] - actor_ctrs[None, :, :]
    dist = jnp.sqrt((diff ** 2).sum(2) + 1e-6)
    mask = dist <= 8.0
    his, wis, cnts = [], [], []
    for w in range(NW):
        lo = w * WROWS
        hi_ = min(N_MAP, lo + WROWS)
        sub = mask[lo:hi_]
        h, wdx = jnp.nonzero(sub, size=SEG, fill_value=0)
        his.append(h.astype(jnp.int32) + lo)
        wis.append(wdx.astype(jnp.int32))
        cnts.append(jnp.count_nonzero(sub).astype(jnp.int32))
    hi = jnp.concatenate(his)
    wi = jnp.concatenate(wis)
    counts = jnp.zeros((NW, 16), jnp.int32).at[:, 0].set(jnp.stack(cnts))
    relx = map_ctrs[hi, 0] - actor_ctrs[wi, 0]
    rely = map_ctrs[hi, 1] - actor_ctrs[wi, 1]
    return hi, wi, relx[:, None], rely[:, None], counts


def _gather_jnp(qc, cc, hi, wi):
    return qc[hi], cc[wi]


def _scatter_jnp(c, hi, counts):
    e = jnp.arange(E_CAP)
    valid = (e % SEG) < counts[e // SEG, 0]
    cm = jnp.where(valid[:, None], c, 0.0)
    p = jax.ops.segment_sum(cm, hi, num_segments=N_MAP)
    return p, jnp.zeros_like(p)


def _att_params_prep(ap):
    return {
        'd0x': ap['dist0_W'][:, 0][None, :],
        'd0y': ap['dist0_W'][:, 1][None, :],
        'd0b': ap['dist0_b'][None, :],
        'w1T': ap['dist1_W'].T,
        'd1g': ap['dist1_g'][None, :],
        'd1b': ap['dist1_b'][None, :],
        'qT': ap['query_W'].T,
        'qg': ap['query_g'][None, :],
        'qb': ap['query_b'][None, :],
        'wdT': ap['ctx0_W'][:, 0:D].T,
        'wqT': ap['ctx0_W'][:, D:2 * D].T,
        'wcT': ap['ctx0_W'][:, 2 * D:3 * D].T,
        'c0g': ap['ctx0_g'][None, :],
        'c0b': ap['ctx0_b'][None, :],
        'c1T': ap['ctx1_W'].T,
        'agtT': ap['agt_W'].T,
        'ng': ap['norm_g'][None, :],
        'nb': ap['norm_b'][None, :],
        'linT': ap['lin_W'].T,
        'lg': ap['lin_g'][None, :],
        'lb': ap['lin_b'][None, :],
    }


def kernel(feat, turn, control, intersect, map_ctrs, actors, actor_ctrs, params):
    meta = jnp.concatenate([turn, control[:, None], intersect[:, None]], axis=1)
    wft = params['meta_W'][:, :D].T
    wmt = params['meta_W'][:, D:D + 4].T
    x = _k0(feat, meta, wft, wmt, params['meta_g'][None, :],
            params['meta_b'][None, :])

    hi, wi, relx, rely, counts = _edge_build_jnp(map_ctrs, actor_ctrs)

    for l in range(2):
        p = _att_params_prep(params['att%d' % l])
        a0, qc, cc = _k1(x, actors, p['agtT'], p['qT'], p['qg'], p['qb'],
                         p['wqT'], p['wcT'])
        qg, cg = _gather_jnp(qc, cc, hi, wi)
        c = _k4(counts, relx, rely, qg, cg, p)
        p0, p1 = _scatter_jnp(c, hi, counts)
        x = _k6(a0, p0, p1, x, p['linT'], p['ng'], p['nb'], p['lg'], p['lb'])
    return x


# speedup vs baseline: 10.1196x; 1.4348x over previous
import functools

import jax
import jax.numpy as jnp
from jax import lax
from jax.experimental import pallas as pl
from jax.experimental.pallas import tpu as pltpu
from jax.experimental.pallas import tpu_sc as plsc

N_MAP = 10000
N_ACT = 1600
D = 128
NW = 32
WROWS = 320
MPAD = NW * WROWS
SEG = 16384
E_CAP = NW * SEG
BLK = 1024
PB = SEG // BLK
RB = 2000

_INTERPRET = False


def _gn(y, g, b):
    m = jnp.mean(y, axis=1, keepdims=True)
    v = jnp.mean((y - m) ** 2, axis=1, keepdims=True)
    return (y - m) / jnp.sqrt(v + 1e-5) * g + b


def _relu(x):
    return jnp.maximum(x, 0.0)


def _k0_body(feat_ref, meta_ref, wft_ref, wmt_ref, g_ref, b_ref, o_ref):
    y = jnp.dot(feat_ref[...], wft_ref[...], preferred_element_type=jnp.float32)
    meta = meta_ref[...]
    wmt = wmt_ref[...]
    for i in range(4):
        y = y + meta[:, i:i + 1] * wmt[i:i + 1, :]
    o_ref[...] = _relu(_gn(y, g_ref[...], b_ref[...]))


def _k0(feat, meta, wft, wmt, g, b):
    grid = (N_MAP // RB,)
    return pl.pallas_call(
        _k0_body,
        grid=grid,
        in_specs=[
            pl.BlockSpec((RB, D), lambda i: (i, 0)),
            pl.BlockSpec((RB, 4), lambda i: (i, 0)),
            pl.BlockSpec((D, D), lambda i: (0, 0)),
            pl.BlockSpec((4, D), lambda i: (0, 0)),
            pl.BlockSpec((1, D), lambda i: (0, 0)),
            pl.BlockSpec((1, D), lambda i: (0, 0)),
        ],
        out_specs=pl.BlockSpec((RB, D), lambda i: (i, 0)),
        out_shape=jax.ShapeDtypeStruct((N_MAP, D), jnp.float32),
        interpret=_INTERPRET,
    )(feat, meta, wft, wmt, g, b)


def _k1_body(x_ref, act_ref, agtT_ref, qT_ref, qg_ref, qb_ref, wqT_ref,
             wcT_ref, a0_ref, qc_ref, cc_ref):
    a0_ref[...] = jnp.dot(x_ref[...], agtT_ref[...],
                          preferred_element_type=jnp.float32)
    q = _relu(_gn(jnp.dot(x_ref[...], qT_ref[...],
                          preferred_element_type=jnp.float32),
                  qg_ref[...], qb_ref[...]))
    qc_ref[...] = jnp.dot(q, wqT_ref[...], preferred_element_type=jnp.float32)

    @pl.when(pl.program_id(0) == 0)
    def _():
        cc_ref[...] = jnp.dot(act_ref[...], wcT_ref[...],
                              preferred_element_type=jnp.float32)


def _k1(x, actors, agtT, qT, qg, qb, wqT, wcT):
    grid = (N_MAP // RB,)
    return pl.pallas_call(
        _k1_body,
        grid=grid,
        in_specs=[
            pl.BlockSpec((RB, D), lambda i: (i, 0)),
            pl.BlockSpec((N_ACT, D), lambda i: (0, 0)),
            pl.BlockSpec((D, D), lambda i: (0, 0)),
            pl.BlockSpec((D, D), lambda i: (0, 0)),
            pl.BlockSpec((1, D), lambda i: (0, 0)),
            pl.BlockSpec((1, D), lambda i: (0, 0)),
            pl.BlockSpec((D, D), lambda i: (0, 0)),
            pl.BlockSpec((D, D), lambda i: (0, 0)),
        ],
        out_specs=[
            pl.BlockSpec((RB, D), lambda i: (i, 0)),
            pl.BlockSpec((RB, D), lambda i: (i, 0)),
            pl.BlockSpec((N_ACT, D), lambda i: (0, 0)),
        ],
        out_shape=[
            jax.ShapeDtypeStruct((N_MAP, D), jnp.float32),
            jax.ShapeDtypeStruct((N_MAP, D), jnp.float32),
            jax.ShapeDtypeStruct((N_ACT, D), jnp.float32),
        ],
        interpret=_INTERPRET,
    )(x, actors, agtT, qT, qg, qb, wqT, wcT)


def _k4_body(cnt_ref, relx_ref, rely_ref, qg_ref, cg_ref,
             d0x_ref, d0y_ref, d0b_ref, w1T_ref, d1g_ref, d1b_ref,
             wdT_ref, c0g_ref, c0b_ref, c1T_ref, o_ref):
    pid = pl.program_id(0)
    s = pid // PB
    base = (pid % PB) * BLK
    cnt = cnt_ref[s, 0]

    @pl.when(base < cnt)
    def _():
        d0 = _relu(relx_ref[...] * d0x_ref[...] + rely_ref[...] * d0y_ref[...]
                   + d0b_ref[...])
        d1 = _relu(_gn(jnp.dot(d0, w1T_ref[...],
                               preferred_element_type=jnp.float32),
                       d1g_ref[...], d1b_ref[...]))
        e = (jnp.dot(d1, wdT_ref[...], preferred_element_type=jnp.float32)
             + qg_ref[...] + cg_ref[...])
        c1 = _relu(_gn(e, c0g_ref[...], c0b_ref[...]))
        c = jnp.dot(c1, c1T_ref[...], preferred_element_type=jnp.float32)
        row = base + lax.broadcasted_iota(jnp.int32, (BLK, 1), 0)
        o_ref[...] = jnp.where(row < cnt, c, 0.0)


def _k4(counts, relx, rely, qg, cg, p):
    grid = (NW * PB,)
    wspec = pl.BlockSpec((D, D), lambda i: (0, 0))
    vspec = pl.BlockSpec((1, D), lambda i: (0, 0))
    espec = pl.BlockSpec((BLK, D), lambda i: (i, 0))
    sspec = pl.BlockSpec((BLK, 1), lambda i: (i, 0))
    return pl.pallas_call(
        _k4_body,
        grid=grid,
        in_specs=[
            pl.BlockSpec(memory_space=pltpu.SMEM),
            sspec, sspec, espec, espec,
            vspec, vspec, vspec, wspec, vspec, vspec,
            wspec, vspec, vspec, wspec,
        ],
        out_specs=espec,
        out_shape=jax.ShapeDtypeStruct((E_CAP, D), jnp.float32),
        interpret=_INTERPRET,
    )(counts, relx, rely, qg, cg,
      p['d0x'], p['d0y'], p['d0b'], p['w1T'], p['d1g'], p['d1b'],
      p['wdT'], p['c0g'], p['c0b'], p['c1T'])


def _k6_body(a0_ref, p0_ref, p1_ref, res_ref, linT_ref,
             ng_ref, nb_ref, lg_ref, lb_ref, o_ref):
    a = a0_ref[...] + p0_ref[...] + p1_ref[...]
    h = _relu(_gn(a, ng_ref[...], nb_ref[...]))
    h2 = _gn(jnp.dot(h, linT_ref[...], preferred_element_type=jnp.float32),
             lg_ref[...], lb_ref[...])
    o_ref[...] = _relu(h2 + res_ref[...])


def _k6(a0, p0, p1, res, linT, ng, nb, lg, lb):
    grid = (N_MAP // RB,)
    rspec = pl.BlockSpec((RB, D), lambda i: (i, 0))
    wspec = pl.BlockSpec((D, D), lambda i: (0, 0))
    vspec = pl.BlockSpec((1, D), lambda i: (0, 0))
    return pl.pallas_call(
        _k6_body,
        grid=grid,
        in_specs=[rspec, rspec, rspec, rspec, wspec,
                  vspec, vspec, vspec, vspec],
        out_specs=rspec,
        out_shape=jax.ShapeDtypeStruct((N_MAP, D), jnp.float32),
        interpret=_INTERPRET,
    )(a0, p0, p1, res, linT, ng, nb, lg, lb)


def _sc_edge_build(mxp, myp, ax, ay):
    mesh = plsc.VectorSubcoreMesh(core_axis_name="c", subcore_axis_name="s")

    @functools.partial(
        pl.kernel,
        out_type=[
            jax.ShapeDtypeStruct((NW, SEG), jnp.int32),
            jax.ShapeDtypeStruct((NW, SEG), jnp.int32),
            jax.ShapeDtypeStruct((NW, SEG), jnp.float32),
            jax.ShapeDtypeStruct((NW, SEG), jnp.float32),
            jax.ShapeDtypeStruct((NW, 16), jnp.int32),
        ],
        mesh=mesh,
        scratch_types=[
            pltpu.VMEM((N_ACT,), jnp.float32),
            pltpu.VMEM((N_ACT,), jnp.float32),
            pltpu.VMEM((WROWS + 16,), jnp.float32),
            pltpu.VMEM((WROWS + 16,), jnp.float32),
            pltpu.VMEM((SEG + 16,), jnp.int32),
            pltpu.VMEM((SEG + 16,), jnp.int32),
            pltpu.VMEM((SEG + 16,), jnp.float32),
            pltpu.VMEM((SEG + 16,), jnp.float32),
            pltpu.VMEM((16,), jnp.int32),
        ],
    )
    def k(mx_h, my_h, ax_h, ay_h, hi_o, wi_o, rx_o, ry_o, cnt_o,
          ax_v, ay_v, mx_v, my_v, hib, wib, rxb, ryb, cntv):
        cid = lax.axis_index("c")
        sid = lax.axis_index("s")
        w = sid * 2 + cid
        row0 = w * WROWS
        nrows = jnp.minimum(WROWS, jnp.maximum(N_MAP - row0, 0))
        pltpu.sync_copy(ax_h, ax_v)
        pltpu.sync_copy(ay_h, ay_v)
        pltpu.sync_copy(mx_h.at[pl.ds(row0, WROWS)], mx_v.at[pl.ds(0, WROWS)])
        pltpu.sync_copy(my_h.at[pl.ds(row0, WROWS)], my_v.at[pl.ds(0, WROWS)])

        zi = jnp.zeros((16,), jnp.int32)

        def zbody(i, _):
            hib[pl.ds(i * 16, 16)] = zi
            wib[pl.ds(i * 16, 16)] = zi
            return 0

        lax.fori_loop(0, (SEG + 16) // 16, zbody, 0)

        lane = lax.iota(jnp.int32, 16)

        def row_body(r, cnt):
            mxs = jnp.full((16,), mx_v[pl.ds(r, 16)][0], jnp.float32)
            mys = jnp.full((16,), my_v[pl.ds(r, 16)][0], jnp.float32)
            hval = jnp.full((16,), row0 + r, jnp.int32)

            def ch_body(a, cnt):
                axv = ax_v[pl.ds(a * 16, 16)]
                ayv = ay_v[pl.ds(a * 16, 16)]
                dx = mxs - axv
                dy = mys - ayv
                d2 = dx * dx + dy * dy
                m = (d2 + 1e-6) <= 64.0
                posf = plsc.cumsum(jnp.where(m, 1.0, 0.0))
                pos = posf.astype(jnp.int32)
                idx = jnp.maximum(jnp.full((16,), cnt, jnp.int32) + pos - 1, 0)
                plsc.store_scatter(hib, [idx], hval, mask=m)
                plsc.store_scatter(wib, [idx],
                                   jnp.full((16,), a * 16, jnp.int32) + lane,
                                   mask=m)
                plsc.store_scatter(rxb, [idx], dx, mask=m)
                plsc.store_scatter(ryb, [idx], dy, mask=m)
                return jnp.minimum(cnt + pos[15], SEG)

            return lax.fori_loop(0, N_ACT // 16, ch_body, cnt)

        cnt = lax.fori_loop(0, nrows, row_body, jnp.int32(0))

        cntv[...] = jnp.full((16,), cnt, jnp.int32)
        pltpu.sync_copy(hib.at[pl.ds(0, SEG)], hi_o.at[w])
        pltpu.sync_copy(wib.at[pl.ds(0, SEG)], wi_o.at[w])
        pltpu.sync_copy(rxb.at[pl.ds(0, SEG)], rx_o.at[w])
        pltpu.sync_copy(ryb.at[pl.ds(0, SEG)], ry_o.at[w])
        pltpu.sync_copy(cntv, cnt_o.at[w])

    return k(mxp, myp, ax, ay)


GCH = 128


def _sc_gather(qc, cc, hi2, wi2, counts):
    mesh = plsc.VectorSubcoreMesh(core_axis_name="c", subcore_axis_name="s")

    @functools.partial(
        pl.kernel,
        out_type=[
            jax.ShapeDtypeStruct((E_CAP, D), jnp.float32),
            jax.ShapeDtypeStruct((E_CAP, D), jnp.float32),
        ],
        mesh=mesh,
        scratch_types=[
            pltpu.VMEM((SEG,), jnp.int32),
            pltpu.VMEM((SEG,), jnp.int32),
            pltpu.VMEM((GCH, D), jnp.float32),
            pltpu.VMEM((GCH, D), jnp.float32),
            pltpu.VMEM((16,), jnp.int32),
        ],
    )
    def k(qc_h, cc_h, hi_h, wi_h, cnt_h, qg_o, cg_o,
          hi_v, wi_v, qbuf, cbuf, cntv):
        cid = lax.axis_index("c")
        sid = lax.axis_index("s")
        w = sid * 2 + cid
        pltpu.sync_copy(hi_h.at[w], hi_v)
        pltpu.sync_copy(wi_h.at[w], wi_v)
        pltpu.sync_copy(cnt_h.at[w], cntv)
        cnt = cntv[...][0]
        nch = (cnt + (GCH - 1)) // GCH

        def body(j, _):
            pltpu.sync_copy(qc_h.at[hi_v.at[pl.ds(j * GCH, GCH)]], qbuf)
            pltpu.sync_copy(cc_h.at[wi_v.at[pl.ds(j * GCH, GCH)]], cbuf)
            base = w * SEG + j * GCH
            pltpu.sync_copy(qbuf, qg_o.at[pl.ds(base, GCH)])
            pltpu.sync_copy(cbuf, cg_o.at[pl.ds(base, GCH)])
            return 0

        lax.fori_loop(0, nch, body, 0)

    return k(qc, cc, hi2, wi2, counts)


SCH = 64
RPS = 632
NMP = 16 * RPS


def _sc_scatter(c, hi3, counts, zrows):
    mesh = plsc.VectorSubcoreMesh(core_axis_name="c", subcore_axis_name="s")

    @functools.partial(
        pl.kernel,
        out_type=[
            jax.ShapeDtypeStruct((NMP, D), jnp.float32),
            jax.ShapeDtypeStruct((NMP, D), jnp.float32),
        ],
        mesh=mesh,
        scratch_types=[
            pltpu.VMEM_SHARED((NMP, D), jnp.float32),
            pltpu.VMEM((SEG // SCH, SCH), jnp.int32),
            pltpu.VMEM((SCH, D), jnp.float32),
            pltpu.VMEM((16,), jnp.int32),
        ],
    )
    def k(c_h, hi_h, cnt_h, z_h, p0_o, p1_o, acc, hiv, cbuf, cntv):
        cid = lax.axis_index("c")
        sid = lax.axis_index("s")
        w = sid * 2 + cid
        pltpu.sync_copy(z_h, acc.at[pl.ds(sid * RPS, RPS)])
        pltpu.sync_copy(hi_h.at[w], hiv)
        pltpu.sync_copy(cnt_h.at[w], cntv)
        cnt = cntv[...][0]
        nch = (cnt + (SCH - 1)) // SCH
        plsc.subcore_barrier()

        def body(j, _):
            pltpu.sync_copy(c_h.at[pl.ds(w * SEG + j * SCH, SCH)], cbuf)
            pltpu.sync_copy(cbuf, acc.at[hiv.at[j]], add=True)
            return 0

        lax.fori_loop(0, nch, body, 0)
        plsc.subcore_barrier()

        @pl.when(cid == 0)
        def _():
            pltpu.sync_copy(acc.at[pl.ds(sid * RPS, RPS)],
                            p0_o.at[pl.ds(sid * RPS, RPS)])

        @pl.when(cid == 1)
        def _():
            pltpu.sync_copy(acc.at[pl.ds(sid * RPS, RPS)],
                            p1_o.at[pl.ds(sid * RPS, RPS)])

    return k(c, hi3, counts, zrows)


def _edge_build_jnp(map_ctrs, actor_ctrs):
    diff = map_ctrs[:, None, :] - actor_ctrs[None, :, :]
    dist = jnp.sqrt((diff ** 2).sum(2) + 1e-6)
    mask = dist <= 8.0
    his, wis, cnts = [], [], []
    for w in range(NW):
        lo = w * WROWS
        hi_ = min(N_MAP, lo + WROWS)
        sub = mask[lo:hi_]
        h, wdx = jnp.nonzero(sub, size=SEG, fill_value=0)
        his.append(h.astype(jnp.int32) + lo)
        wis.append(wdx.astype(jnp.int32))
        cnts.append(jnp.count_nonzero(sub).astype(jnp.int32))
    hi = jnp.concatenate(his)
    wi = jnp.concatenate(wis)
    counts = jnp.zeros((NW, 16), jnp.int32).at[:, 0].set(jnp.stack(cnts))
    relx = map_ctrs[hi, 0] - actor_ctrs[wi, 0]
    rely = map_ctrs[hi, 1] - actor_ctrs[wi, 1]
    return hi, wi, relx[:, None], rely[:, None], counts


def _gather_jnp(qc, cc, hi, wi):
    return qc[hi], cc[wi]


def _scatter_jnp(c, hi, counts):
    e = jnp.arange(E_CAP)
    valid = (e % SEG) < counts[e // SEG, 0]
    cm = jnp.where(valid[:, None], c, 0.0)
    p = jax.ops.segment_sum(cm, hi, num_segments=N_MAP)
    return p, jnp.zeros_like(p)


def _att_params_prep(ap):
    return {
        'd0x': ap['dist0_W'][:, 0][None, :],
        'd0y': ap['dist0_W'][:, 1][None, :],
        'd0b': ap['dist0_b'][None, :],
        'w1T': ap['dist1_W'].T,
        'd1g': ap['dist1_g'][None, :],
        'd1b': ap['dist1_b'][None, :],
        'qT': ap['query_W'].T,
        'qg': ap['query_g'][None, :],
        'qb': ap['query_b'][None, :],
        'wdT': ap['ctx0_W'][:, 0:D].T,
        'wqT': ap['ctx0_W'][:, D:2 * D].T,
        'wcT': ap['ctx0_W'][:, 2 * D:3 * D].T,
        'c0g': ap['ctx0_g'][None, :],
        'c0b': ap['ctx0_b'][None, :],
        'c1T': ap['ctx1_W'].T,
        'agtT': ap['agt_W'].T,
        'ng': ap['norm_g'][None, :],
        'nb': ap['norm_b'][None, :],
        'linT': ap['lin_W'].T,
        'lg': ap['lin_g'][None, :],
        'lb': ap['lin_b'][None, :],
    }


def kernel(feat, turn, control, intersect, map_ctrs, actors, actor_ctrs, params):
    meta = jnp.concatenate([turn, control[:, None], intersect[:, None]], axis=1)
    wft = params['meta_W'][:, :D].T
    wmt = params['meta_W'][:, D:D + 4].T
    x = _k0(feat, meta, wft, wmt, params['meta_g'][None, :],
            params['meta_b'][None, :])

    hi, wi, relx, rely, counts = _edge_build_jnp(map_ctrs, actor_ctrs)
    hi2 = hi.reshape(NW, SEG)
    wi2 = wi.reshape(NW, SEG)
    hi3 = hi.reshape(NW, SEG // SCH, SCH)
    zrows = jnp.zeros((RPS, D), jnp.float32)

    for l in range(2):
        p = _att_params_prep(params['att%d' % l])
        a0, qc, cc = _k1(x, actors, p['agtT'], p['qT'], p['qg'], p['qb'],
                         p['wqT'], p['wcT'])
        qg, cg = _sc_gather(qc, cc, hi2, wi2, counts)
        c = _k4(counts, relx, rely, qg, cg, p)
        p0f, p1f = _sc_scatter(c, hi3, counts, zrows)
        p0 = p0f[:N_MAP]
        p1 = p1f[:N_MAP]
        x = _k6(a0, p0, p1, x, p['linT'], p['ng'], p['nb'], p['lg'], p['lb'])
    return x


# speedup vs baseline: 65.1735x; 6.4403x over previous
import functools

import jax
import jax.numpy as jnp
from jax import lax
from jax.experimental import pallas as pl
from jax.experimental.pallas import tpu as pltpu
from jax.experimental.pallas import tpu_sc as plsc

N_MAP = 10000
N_ACT = 1600
D = 128
NW = 32
WROWS = 320
MPAD = NW * WROWS
SEG = 16384
E_CAP = NW * SEG
BLK = 1024
PB = SEG // BLK
RB = 2000

_INTERPRET = False


def _gn(y, g, b):
    m = jnp.mean(y, axis=1, keepdims=True)
    v = jnp.mean((y - m) ** 2, axis=1, keepdims=True)
    return (y - m) / jnp.sqrt(v + 1e-5) * g + b


def _relu(x):
    return jnp.maximum(x, 0.0)


def _k0_body(feat_ref, meta_ref, wft_ref, wmt_ref, g_ref, b_ref, o_ref):
    y = jnp.dot(feat_ref[...], wft_ref[...], preferred_element_type=jnp.float32)
    meta = meta_ref[...]
    wmt = wmt_ref[...]
    for i in range(4):
        y = y + meta[:, i:i + 1] * wmt[i:i + 1, :]
    o_ref[...] = _relu(_gn(y, g_ref[...], b_ref[...]))


def _k0(feat, meta, wft, wmt, g, b):
    grid = (N_MAP // RB,)
    return pl.pallas_call(
        _k0_body,
        grid=grid,
        in_specs=[
            pl.BlockSpec((RB, D), lambda i: (i, 0)),
            pl.BlockSpec((RB, 4), lambda i: (i, 0)),
            pl.BlockSpec((D, D), lambda i: (0, 0)),
            pl.BlockSpec((4, D), lambda i: (0, 0)),
            pl.BlockSpec((1, D), lambda i: (0, 0)),
            pl.BlockSpec((1, D), lambda i: (0, 0)),
        ],
        out_specs=pl.BlockSpec((RB, D), lambda i: (i, 0)),
        out_shape=jax.ShapeDtypeStruct((N_MAP, D), jnp.float32),
        interpret=_INTERPRET,
    )(feat, meta, wft, wmt, g, b)


def _k1_body(x_ref, act_ref, agtT_ref, qT_ref, qg_ref, qb_ref, wqT_ref,
             wcT_ref, a0_ref, qc_ref, cc_ref):
    a0_ref[...] = jnp.dot(x_ref[...], agtT_ref[...],
                          preferred_element_type=jnp.float32)
    q = _relu(_gn(jnp.dot(x_ref[...], qT_ref[...],
                          preferred_element_type=jnp.float32),
                  qg_ref[...], qb_ref[...]))
    qc_ref[...] = jnp.dot(q, wqT_ref[...], preferred_element_type=jnp.float32)

    @pl.when(pl.program_id(0) == 0)
    def _():
        cc_ref[...] = jnp.dot(act_ref[...], wcT_ref[...],
                              preferred_element_type=jnp.float32)


def _k1(x, actors, agtT, qT, qg, qb, wqT, wcT):
    grid = (N_MAP // RB,)
    return pl.pallas_call(
        _k1_body,
        grid=grid,
        in_specs=[
            pl.BlockSpec((RB, D), lambda i: (i, 0)),
            pl.BlockSpec((N_ACT, D), lambda i: (0, 0)),
            pl.BlockSpec((D, D), lambda i: (0, 0)),
            pl.BlockSpec((D, D), lambda i: (0, 0)),
            pl.BlockSpec((1, D), lambda i: (0, 0)),
            pl.BlockSpec((1, D), lambda i: (0, 0)),
            pl.BlockSpec((D, D), lambda i: (0, 0)),
            pl.BlockSpec((D, D), lambda i: (0, 0)),
        ],
        out_specs=[
            pl.BlockSpec((RB, D), lambda i: (i, 0)),
            pl.BlockSpec((RB, D), lambda i: (i, 0)),
            pl.BlockSpec((N_ACT, D), lambda i: (0, 0)),
        ],
        out_shape=[
            jax.ShapeDtypeStruct((N_MAP, D), jnp.float32),
            jax.ShapeDtypeStruct((N_MAP, D), jnp.float32),
            jax.ShapeDtypeStruct((N_ACT, D), jnp.float32),
        ],
        interpret=_INTERPRET,
    )(x, actors, agtT, qT, qg, qb, wqT, wcT)


def _k4_body(cnt_ref, relx_ref, rely_ref, qg_ref, cg_ref,
             d0x_ref, d0y_ref, d0b_ref, w1T_ref, d1g_ref, d1b_ref,
             wdT_ref, c0g_ref, c0b_ref, c1T_ref, o_ref):
    pid = pl.program_id(0)
    s = pid // PB
    base = (pid % PB) * BLK
    cnt = cnt_ref[s, 0]

    @pl.when(base < cnt)
    def _():
        d0 = _relu(relx_ref[...] * d0x_ref[...] + rely_ref[...] * d0y_ref[...]
                   + d0b_ref[...])
        d1 = _relu(_gn(jnp.dot(d0, w1T_ref[...],
                               preferred_element_type=jnp.float32),
                       d1g_ref[...], d1b_ref[...]))
        e = (jnp.dot(d1, wdT_ref[...], preferred_element_type=jnp.float32)
             + qg_ref[...] + cg_ref[...])
        c1 = _relu(_gn(e, c0g_ref[...], c0b_ref[...]))
        c = jnp.dot(c1, c1T_ref[...], preferred_element_type=jnp.float32)
        row = base + lax.broadcasted_iota(jnp.int32, (BLK, 1), 0)
        o_ref[...] = jnp.where(row < cnt, c, 0.0)


def _k4(counts, relx, rely, qg, cg, p):
    grid = (NW * PB,)
    wspec = pl.BlockSpec((D, D), lambda i: (0, 0))
    vspec = pl.BlockSpec((1, D), lambda i: (0, 0))
    espec = pl.BlockSpec((BLK, D), lambda i: (i, 0))
    sspec = pl.BlockSpec((BLK, 1), lambda i: (i, 0))
    return pl.pallas_call(
        _k4_body,
        grid=grid,
        in_specs=[
            pl.BlockSpec(memory_space=pltpu.SMEM),
            sspec, sspec, espec, espec,
            vspec, vspec, vspec, wspec, vspec, vspec,
            wspec, vspec, vspec, wspec,
        ],
        out_specs=espec,
        out_shape=jax.ShapeDtypeStruct((E_CAP, D), jnp.float32),
        interpret=_INTERPRET,
    )(counts, relx, rely, qg, cg,
      p['d0x'], p['d0y'], p['d0b'], p['w1T'], p['d1g'], p['d1b'],
      p['wdT'], p['c0g'], p['c0b'], p['c1T'])


def _k6_body(a0_ref, p0_ref, p1_ref, res_ref, linT_ref,
             ng_ref, nb_ref, lg_ref, lb_ref, o_ref):
    a = a0_ref[...] + p0_ref[...] + p1_ref[...]
    h = _relu(_gn(a, ng_ref[...], nb_ref[...]))
    h2 = _gn(jnp.dot(h, linT_ref[...], preferred_element_type=jnp.float32),
             lg_ref[...], lb_ref[...])
    o_ref[...] = _relu(h2 + res_ref[...])


def _k6(a0, p0, p1, res, linT, ng, nb, lg, lb):
    grid = (N_MAP // RB,)
    rspec = pl.BlockSpec((RB, D), lambda i: (i, 0))
    wspec = pl.BlockSpec((D, D), lambda i: (0, 0))
    vspec = pl.BlockSpec((1, D), lambda i: (0, 0))
    return pl.pallas_call(
        _k6_body,
        grid=grid,
        in_specs=[rspec, rspec, rspec, rspec, wspec,
                  vspec, vspec, vspec, vspec],
        out_specs=rspec,
        out_shape=jax.ShapeDtypeStruct((N_MAP, D), jnp.float32),
        interpret=_INTERPRET,
    )(a0, p0, p1, res, linT, ng, nb, lg, lb)


def _sc_edge_build(mxp, myp, ax, ay):
    mesh = plsc.VectorSubcoreMesh(core_axis_name="c", subcore_axis_name="s")

    @functools.partial(
        pl.kernel,
        out_type=[
            jax.ShapeDtypeStruct((NW, SEG), jnp.int32),
            jax.ShapeDtypeStruct((NW, SEG), jnp.int32),
            jax.ShapeDtypeStruct((NW, SEG), jnp.float32),
            jax.ShapeDtypeStruct((NW, SEG), jnp.float32),
            jax.ShapeDtypeStruct((NW, 16), jnp.int32),
        ],
        mesh=mesh,
        scratch_types=[
            pltpu.VMEM((N_ACT,), jnp.float32),
            pltpu.VMEM((N_ACT,), jnp.float32),
            pltpu.VMEM((WROWS + 16,), jnp.float32),
            pltpu.VMEM((WROWS + 16,), jnp.float32),
            pltpu.VMEM((SEG + 16,), jnp.int32),
            pltpu.VMEM((SEG + 16,), jnp.int32),
            pltpu.VMEM((SEG + 16,), jnp.float32),
            pltpu.VMEM((SEG + 16,), jnp.float32),
            pltpu.VMEM((16,), jnp.int32),
            pltpu.VMEM((48,), jnp.int32),
        ],
    )
    def k(mx_h, my_h, ax_h, ay_h, hi_o, wi_o, rx_o, ry_o, cnt_o,
          ax_v, ay_v, mx_v, my_v, hib, wib, rxb, ryb, cntv, pbuf):
        cid = lax.axis_index("c")
        sid = lax.axis_index("s")
        w = sid * 2 + cid
        row0 = w * WROWS
        nrows = jnp.minimum(WROWS, jnp.maximum(N_MAP - row0, 0))
        pltpu.sync_copy(ax_h, ax_v)
        pltpu.sync_copy(ay_h, ay_v)
        pltpu.sync_copy(mx_h.at[pl.ds(row0, WROWS)], mx_v.at[pl.ds(0, WROWS)])
        pltpu.sync_copy(my_h.at[pl.ds(row0, WROWS)], my_v.at[pl.ds(0, WROWS)])

        zi = jnp.zeros((16,), jnp.int32)

        def zbody(i, _):
            hib[pl.ds(i * 16, 16)] = zi
            wib[pl.ds(i * 16, 16)] = zi
            return 0

        lax.fori_loop(0, (SEG + 16) // 16, zbody, 0)

        lane = lax.iota(jnp.int32, 16)
        zi16 = jnp.zeros((16,), jnp.int32)
        zf16 = jnp.zeros((16,), jnp.float32)
        pbuf[pl.ds(0, 16)] = zi16
        pbuf[pl.ds(32, 16)] = zi16

        def row_body(r, cnt):
            mxs = jnp.full((16,), mx_v[pl.ds(r, 16)][0], jnp.float32)
            mys = jnp.full((16,), my_v[pl.ds(r, 16)][0], jnp.float32)
            hval = jnp.full((16,), row0 + r, jnp.int32)

            def ch_body(a, cnt):
                axv = ax_v[pl.ds(a * 16, 16)]
                ayv = ay_v[pl.ds(a * 16, 16)]
                dx = mxs - axv
                dy = mys - ayv
                d2 = dx * dx + dy * dy
                m = (d2 + 1e-6) <= 64.0
                sv = jnp.where(m, 1, 0)
                s = sv
                pbuf[pl.ds(16, 16)] = s
                s = s + pbuf[pl.ds(15, 16)]
                pbuf[pl.ds(16, 16)] = s
                s = s + pbuf[pl.ds(14, 16)]
                pbuf[pl.ds(16, 16)] = s
                s = s + pbuf[pl.ds(12, 16)]
                pbuf[pl.ds(16, 16)] = s
                pos = s + pbuf[pl.ds(8, 16)]
                n = pos[15]

                def do(cnt):
                    owi = zi16
                    odx = zf16
                    ody = zf16
                    for j in range(16):
                        pj = jnp.full((16,), pos[j] - 1, jnp.int32)
                        mj = jnp.full((16,), sv[j], jnp.int32)
                        eqi = jnp.where(lane == pj, mj, 0)
                        eqf = eqi.astype(jnp.float32)
                        owi = owi + eqi * jnp.full((16,), a * 16 + j,
                                                   jnp.int32)
                        odx = odx + eqf * jnp.full((16,), dx[j], jnp.float32)
                        ody = ody + eqf * jnp.full((16,), dy[j], jnp.float32)
                    hib[pl.ds(cnt, 16)] = hval
                    wib[pl.ds(cnt, 16)] = owi
                    rxb[pl.ds(cnt, 16)] = odx
                    ryb[pl.ds(cnt, 16)] = ody
                    return jnp.minimum(cnt + n, SEG)

                return lax.cond(n > 0, do, lambda c: c, cnt)

            return lax.fori_loop(0, N_ACT // 16, ch_body, cnt)

        cnt = lax.fori_loop(0, nrows, row_body, jnp.int32(0))

        cntv[...] = jnp.full((16,), cnt, jnp.int32)
        pltpu.sync_copy(hib.at[pl.ds(0, SEG)], hi_o.at[w])
        pltpu.sync_copy(wib.at[pl.ds(0, SEG)], wi_o.at[w])
        pltpu.sync_copy(rxb.at[pl.ds(0, SEG)], rx_o.at[w])
        pltpu.sync_copy(ryb.at[pl.ds(0, SEG)], ry_o.at[w])
        pltpu.sync_copy(cntv, cnt_o.at[w])

    return k(mxp, myp, ax, ay)


GCH = 128


def _sc_gather(qc, cc, hi2, wi2, counts):
    mesh = plsc.VectorSubcoreMesh(core_axis_name="c", subcore_axis_name="s")

    @functools.partial(
        pl.kernel,
        out_type=[
            jax.ShapeDtypeStruct((E_CAP, D), jnp.float32),
            jax.ShapeDtypeStruct((E_CAP, D), jnp.float32),
        ],
        mesh=mesh,
        scratch_types=[
            pltpu.VMEM((SEG,), jnp.int32),
            pltpu.VMEM((SEG,), jnp.int32),
            pltpu.VMEM((GCH, D), jnp.float32),
            pltpu.VMEM((GCH, D), jnp.float32),
            pltpu.VMEM((16,), jnp.int32),
        ],
    )
    def k(qc_h, cc_h, hi_h, wi_h, cnt_h, qg_o, cg_o,
          hi_v, wi_v, qbuf, cbuf, cntv):
        cid = lax.axis_index("c")
        sid = lax.axis_index("s")
        w = sid * 2 + cid
        pltpu.sync_copy(hi_h.at[w], hi_v)
        pltpu.sync_copy(wi_h.at[w], wi_v)
        pltpu.sync_copy(cnt_h.at[w], cntv)
        cnt = cntv[...][0]
        nch = (cnt + (GCH - 1)) // GCH

        def body(j, _):
            pltpu.sync_copy(qc_h.at[hi_v.at[pl.ds(j * GCH, GCH)]], qbuf)
            pltpu.sync_copy(cc_h.at[wi_v.at[pl.ds(j * GCH, GCH)]], cbuf)
            base = w * SEG + j * GCH
            pltpu.sync_copy(qbuf, qg_o.at[pl.ds(base, GCH)])
            pltpu.sync_copy(cbuf, cg_o.at[pl.ds(base, GCH)])
            return 0

        lax.fori_loop(0, nch, body, 0)

    return k(qc, cc, hi2, wi2, counts)


SCH = 64
RPS = 632
NMP = 16 * RPS


def _sc_scatter(c, hi3, counts, zrows):
    mesh = plsc.VectorSubcoreMesh(core_axis_name="c", subcore_axis_name="s")

    @functools.partial(
        pl.kernel,
        out_type=[
            jax.ShapeDtypeStruct((NMP, D), jnp.float32),
            jax.ShapeDtypeStruct((NMP, D), jnp.float32),
        ],
        mesh=mesh,
        scratch_types=[
            pltpu.VMEM_SHARED((NMP, D), jnp.float32),
            pltpu.VMEM((SEG // SCH, SCH), jnp.int32),
            pltpu.VMEM((SCH, D), jnp.float32),
            pltpu.VMEM((16,), jnp.int32),
        ],
    )
    def k(c_h, hi_h, cnt_h, z_h, p0_o, p1_o, acc, hiv, cbuf, cntv):
        cid = lax.axis_index("c")
        sid = lax.axis_index("s")
        w = sid * 2 + cid
        pltpu.sync_copy(z_h, acc.at[pl.ds(sid * RPS, RPS)])
        pltpu.sync_copy(hi_h.at[w], hiv)
        pltpu.sync_copy(cnt_h.at[w], cntv)
        cnt = cntv[...][0]
        nch = (cnt + (SCH - 1)) // SCH
        plsc.subcore_barrier()

        def body(j, _):
            pltpu.sync_copy(c_h.at[pl.ds(w * SEG + j * SCH, SCH)], cbuf)
            pltpu.sync_copy(cbuf, acc.at[hiv.at[j]], add=True)
            return 0

        lax.fori_loop(0, nch, body, 0)
        plsc.subcore_barrier()

        @pl.when(cid == 0)
        def _():
            pltpu.sync_copy(acc.at[pl.ds(sid * RPS, RPS)],
                            p0_o.at[pl.ds(sid * RPS, RPS)])

        @pl.when(cid == 1)
        def _():
            pltpu.sync_copy(acc.at[pl.ds(sid * RPS, RPS)],
                            p1_o.at[pl.ds(sid * RPS, RPS)])

    return k(c, hi3, counts, zrows)


def _edge_build_jnp(map_ctrs, actor_ctrs):
    diff = map_ctrs[:, None, :] - actor_ctrs[None, :, :]
    dist = jnp.sqrt((diff ** 2).sum(2) + 1e-6)
    mask = dist <= 8.0
    his, wis, cnts = [], [], []
    for w in range(NW):
        lo = w * WROWS
        hi_ = min(N_MAP, lo + WROWS)
        sub = mask[lo:hi_]
        h, wdx = jnp.nonzero(sub, size=SEG, fill_value=0)
        his.append(h.astype(jnp.int32) + lo)
        wis.append(wdx.astype(jnp.int32))
        cnts.append(jnp.count_nonzero(sub).astype(jnp.int32))
    hi = jnp.concatenate(his)
    wi = jnp.concatenate(wis)
    counts = jnp.zeros((NW, 16), jnp.int32).at[:, 0].set(jnp.stack(cnts))
    relx = map_ctrs[hi, 0] - actor_ctrs[wi, 0]
    rely = map_ctrs[hi, 1] - actor_ctrs[wi, 1]
    return hi, wi, relx[:, None], rely[:, None], counts


def _gather_jnp(qc, cc, hi, wi):
    return qc[hi], cc[wi]


def _scatter_jnp(c, hi, counts):
    e = jnp.arange(E_CAP)
    valid = (e % SEG) < counts[e // SEG, 0]
    cm = jnp.where(valid[:, None], c, 0.0)
    p = jax.ops.segment_sum(cm, hi, num_segments=N_MAP)
    return p, jnp.zeros_like(p)


def _att_params_prep(ap):
    return {
        'd0x': ap['dist0_W'][:, 0][None, :],
        'd0y': ap['dist0_W'][:, 1][None, :],
        'd0b': ap['dist0_b'][None, :],
        'w1T': ap['dist1_W'].T,
        'd1g': ap['dist1_g'][None, :],
        'd1b': ap['dist1_b'][None, :],
        'qT': ap['query_W'].T,
        'qg': ap['query_g'][None, :],
        'qb': ap['query_b'][None, :],
        'wdT': ap['ctx0_W'][:, 0:D].T,
        'wqT': ap['ctx0_W'][:, D:2 * D].T,
        'wcT': ap['ctx0_W'][:, 2 * D:3 * D].T,
        'c0g': ap['ctx0_g'][None, :],
        'c0b': ap['ctx0_b'][None, :],
        'c1T': ap['ctx1_W'].T,
        'agtT': ap['agt_W'].T,
        'ng': ap['norm_g'][None, :],
        'nb': ap['norm_b'][None, :],
        'linT': ap['lin_W'].T,
        'lg': ap['lin_g'][None, :],
        'lb': ap['lin_b'][None, :],
    }


def kernel(feat, turn, control, intersect, map_ctrs, actors, actor_ctrs, params):
    meta = jnp.concatenate([turn, control[:, None], intersect[:, None]], axis=1)
    wft = params['meta_W'][:, :D].T
    wmt = params['meta_W'][:, D:D + 4].T
    x = _k0(feat, meta, wft, wmt, params['meta_g'][None, :],
            params['meta_b'][None, :])

    mxp = jnp.zeros((MPAD,), jnp.float32).at[:N_MAP].set(map_ctrs[:, 0])
    myp = jnp.zeros((MPAD,), jnp.float32).at[:N_MAP].set(map_ctrs[:, 1])
    hi2, wi2, rx2, ry2, counts = _sc_edge_build(
        mxp, myp, actor_ctrs[:, 0], actor_ctrs[:, 1])
    hi = hi2.reshape(E_CAP)
    relx = rx2.reshape(E_CAP, 1)
    rely = ry2.reshape(E_CAP, 1)
    hi3 = hi.reshape(NW, SEG // SCH, SCH)
    zrows = jnp.zeros((RPS, D), jnp.float32)

    for l in range(2):
        p = _att_params_prep(params['att%d' % l])
        a0, qc, cc = _k1(x, actors, p['agtT'], p['qT'], p['qg'], p['qb'],
                         p['wqT'], p['wcT'])
        qg, cg = _sc_gather(qc, cc, hi2, wi2, counts)
        c = _k4(counts, relx, rely, qg, cg, p)
        p0f, p1f = _sc_scatter(c, hi3, counts, zrows)
        p0 = p0f[:N_MAP]
        p1 = p1f[:N_MAP]
        x = _k6(a0, p0, p1, x, p['linT'], p['ng'], p['nb'], p['lg'], p['lb'])
    return x


# speedup vs baseline: 65.1765x; 1.0000x over previous
import functools

import jax
import jax.numpy as jnp
from jax import lax
from jax.experimental import pallas as pl
from jax.experimental.pallas import tpu as pltpu
from jax.experimental.pallas import tpu_sc as plsc

N_MAP = 10000
N_ACT = 1600
D = 128
NW = 32
WROWS = 320
MPAD = NW * WROWS
SEG = 16384
E_CAP = NW * SEG
BLK = 1024
PB = SEG // BLK
RB = 2000

_INTERPRET = False


def _gn(y, g, b):
    m = jnp.mean(y, axis=1, keepdims=True)
    v = jnp.mean((y - m) ** 2, axis=1, keepdims=True)
    return (y - m) / jnp.sqrt(v + 1e-5) * g + b


def _relu(x):
    return jnp.maximum(x, 0.0)


def _k0_body(feat_ref, meta_ref, wft_ref, wmt_ref, g_ref, b_ref, o_ref):
    y = jnp.dot(feat_ref[...], wft_ref[...], preferred_element_type=jnp.float32)
    meta = meta_ref[...]
    wmt = wmt_ref[...]
    for i in range(4):
        y = y + meta[:, i:i + 1] * wmt[i:i + 1, :]
    o_ref[...] = _relu(_gn(y, g_ref[...], b_ref[...]))


def _k0(feat, meta, wft, wmt, g, b):
    grid = (N_MAP // RB,)
    return pl.pallas_call(
        _k0_body,
        grid=grid,
        in_specs=[
            pl.BlockSpec((RB, D), lambda i: (i, 0)),
            pl.BlockSpec((RB, 4), lambda i: (i, 0)),
            pl.BlockSpec((D, D), lambda i: (0, 0)),
            pl.BlockSpec((4, D), lambda i: (0, 0)),
            pl.BlockSpec((1, D), lambda i: (0, 0)),
            pl.BlockSpec((1, D), lambda i: (0, 0)),
        ],
        out_specs=pl.BlockSpec((RB, D), lambda i: (i, 0)),
        out_shape=jax.ShapeDtypeStruct((N_MAP, D), jnp.float32),
        interpret=_INTERPRET,
    )(feat, meta, wft, wmt, g, b)


def _k1_body(x_ref, act_ref, agtT_ref, qT_ref, qg_ref, qb_ref, wqT_ref,
             wcT_ref, a0_ref, qc_ref, cc_ref):
    a0_ref[...] = jnp.dot(x_ref[...], agtT_ref[...],
                          preferred_element_type=jnp.float32)
    q = _relu(_gn(jnp.dot(x_ref[...], qT_ref[...],
                          preferred_element_type=jnp.float32),
                  qg_ref[...], qb_ref[...]))
    qc_ref[...] = jnp.dot(q, wqT_ref[...], preferred_element_type=jnp.float32)

    @pl.when(pl.program_id(0) == 0)
    def _():
        cc_ref[...] = jnp.dot(act_ref[...], wcT_ref[...],
                              preferred_element_type=jnp.float32)


def _k1(x, actors, agtT, qT, qg, qb, wqT, wcT):
    grid = (N_MAP // RB,)
    return pl.pallas_call(
        _k1_body,
        grid=grid,
        in_specs=[
            pl.BlockSpec((RB, D), lambda i: (i, 0)),
            pl.BlockSpec((N_ACT, D), lambda i: (0, 0)),
            pl.BlockSpec((D, D), lambda i: (0, 0)),
            pl.BlockSpec((D, D), lambda i: (0, 0)),
            pl.BlockSpec((1, D), lambda i: (0, 0)),
            pl.BlockSpec((1, D), lambda i: (0, 0)),
            pl.BlockSpec((D, D), lambda i: (0, 0)),
            pl.BlockSpec((D, D), lambda i: (0, 0)),
        ],
        out_specs=[
            pl.BlockSpec((RB, D), lambda i: (i, 0)),
            pl.BlockSpec((RB, D), lambda i: (i, 0)),
            pl.BlockSpec((N_ACT, D), lambda i: (0, 0)),
        ],
        out_shape=[
            jax.ShapeDtypeStruct((N_MAP, D), jnp.float32),
            jax.ShapeDtypeStruct((N_MAP, D), jnp.float32),
            jax.ShapeDtypeStruct((N_ACT, D), jnp.float32),
        ],
        interpret=_INTERPRET,
    )(x, actors, agtT, qT, qg, qb, wqT, wcT)


def _k4_body(cnt_ref, relx_ref, rely_ref, qg_ref, cg_ref,
             d0x_ref, d0y_ref, d0b_ref, w1T_ref, d1g_ref, d1b_ref,
             wdT_ref, c0g_ref, c0b_ref, c1T_ref, o_ref):
    pid = pl.program_id(0)
    s = pid // PB
    base = (pid % PB) * BLK
    cnt = cnt_ref[s, 0]

    @pl.when(base < cnt)
    def _():
        d0 = _relu(relx_ref[...] * d0x_ref[...] + rely_ref[...] * d0y_ref[...]
                   + d0b_ref[...])
        d1 = _relu(_gn(jnp.dot(d0, w1T_ref[...],
                               preferred_element_type=jnp.float32),
                       d1g_ref[...], d1b_ref[...]))
        e = (jnp.dot(d1, wdT_ref[...], preferred_element_type=jnp.float32)
             + qg_ref[...] + cg_ref[...])
        c1 = _relu(_gn(e, c0g_ref[...], c0b_ref[...]))
        c = jnp.dot(c1, c1T_ref[...], preferred_element_type=jnp.float32)
        row = base + lax.broadcasted_iota(jnp.int32, (BLK, 1), 0)
        o_ref[...] = jnp.where(row < cnt, c, 0.0)


def _k4(counts, relx, rely, qg, cg, p):
    grid = (NW * PB,)
    wspec = pl.BlockSpec((D, D), lambda i: (0, 0))
    vspec = pl.BlockSpec((1, D), lambda i: (0, 0))
    espec = pl.BlockSpec((BLK, D), lambda i: (i, 0))
    sspec = pl.BlockSpec((BLK, 1), lambda i: (i, 0))
    return pl.pallas_call(
        _k4_body,
        grid=grid,
        in_specs=[
            pl.BlockSpec(memory_space=pltpu.SMEM),
            sspec, sspec, espec, espec,
            vspec, vspec, vspec, wspec, vspec, vspec,
            wspec, vspec, vspec, wspec,
        ],
        out_specs=espec,
        out_shape=jax.ShapeDtypeStruct((E_CAP, D), jnp.float32),
        interpret=_INTERPRET,
    )(counts, relx, rely, qg, cg,
      p['d0x'], p['d0y'], p['d0b'], p['w1T'], p['d1g'], p['d1b'],
      p['wdT'], p['c0g'], p['c0b'], p['c1T'])


def _k6_body(a0_ref, p0_ref, p1_ref, res_ref, linT_ref,
             ng_ref, nb_ref, lg_ref, lb_ref, o_ref):
    a = a0_ref[...] + p0_ref[...] + p1_ref[...]
    h = _relu(_gn(a, ng_ref[...], nb_ref[...]))
    h2 = _gn(jnp.dot(h, linT_ref[...], preferred_element_type=jnp.float32),
             lg_ref[...], lb_ref[...])
    o_ref[...] = _relu(h2 + res_ref[...])


def _k6(a0, p0, p1, res, linT, ng, nb, lg, lb):
    grid = (N_MAP // RB,)
    rspec = pl.BlockSpec((RB, D), lambda i: (i, 0))
    wspec = pl.BlockSpec((D, D), lambda i: (0, 0))
    vspec = pl.BlockSpec((1, D), lambda i: (0, 0))
    return pl.pallas_call(
        _k6_body,
        grid=grid,
        in_specs=[rspec, rspec, rspec, rspec, wspec,
                  vspec, vspec, vspec, vspec],
        out_specs=rspec,
        out_shape=jax.ShapeDtypeStruct((N_MAP, D), jnp.float32),
        interpret=_INTERPRET,
    )(a0, p0, p1, res, linT, ng, nb, lg, lb)


def _sc_edge_build(mxp, myp, ax, ay):
    mesh = plsc.VectorSubcoreMesh(core_axis_name="c", subcore_axis_name="s")

    @functools.partial(
        pl.kernel,
        out_type=[
            jax.ShapeDtypeStruct((NW, SEG), jnp.int32),
            jax.ShapeDtypeStruct((NW, SEG), jnp.int32),
            jax.ShapeDtypeStruct((NW, SEG), jnp.float32),
            jax.ShapeDtypeStruct((NW, SEG), jnp.float32),
            jax.ShapeDtypeStruct((NW, 16), jnp.int32),
        ],
        mesh=mesh,
        scratch_types=[
            pltpu.VMEM((N_ACT,), jnp.float32),
            pltpu.VMEM((N_ACT,), jnp.float32),
            pltpu.VMEM((WROWS + 16,), jnp.float32),
            pltpu.VMEM((WROWS + 16,), jnp.float32),
            pltpu.VMEM((SEG + 16,), jnp.int32),
            pltpu.VMEM((SEG + 16,), jnp.int32),
            pltpu.VMEM((SEG + 16,), jnp.float32),
            pltpu.VMEM((SEG + 16,), jnp.float32),
            pltpu.VMEM((16,), jnp.int32),
            pltpu.VMEM((48,), jnp.int32),
        ],
    )
    def k(mx_h, my_h, ax_h, ay_h, hi_o, wi_o, rx_o, ry_o, cnt_o,
          ax_v, ay_v, mx_v, my_v, hib, wib, rxb, ryb, cntv, pbuf):
        cid = lax.axis_index("c")
        sid = lax.axis_index("s")
        w = sid * 2 + cid
        row0 = w * WROWS
        nrows = jnp.minimum(WROWS, jnp.maximum(N_MAP - row0, 0))
        pltpu.sync_copy(ax_h, ax_v)
        pltpu.sync_copy(ay_h, ay_v)
        pltpu.sync_copy(mx_h.at[pl.ds(row0, WROWS)], mx_v.at[pl.ds(0, WROWS)])
        pltpu.sync_copy(my_h.at[pl.ds(row0, WROWS)], my_v.at[pl.ds(0, WROWS)])

        zi = jnp.zeros((16,), jnp.int32)

        def zbody(i, _):
            hib[pl.ds(i * 16, 16)] = zi
            wib[pl.ds(i * 16, 16)] = zi
            return 0

        lax.fori_loop(0, (SEG + 16) // 16, zbody, 0)

        lane = lax.iota(jnp.int32, 16)
        zi16 = jnp.zeros((16,), jnp.int32)
        zf16 = jnp.zeros((16,), jnp.float32)
        pbuf[pl.ds(0, 16)] = zi16
        pbuf[pl.ds(32, 16)] = zi16

        def row_body(r, cnt):
            mxs = jnp.full((16,), mx_v[pl.ds(r, 16)][0], jnp.float32)
            mys = jnp.full((16,), my_v[pl.ds(r, 16)][0], jnp.float32)
            hval = jnp.full((16,), row0 + r, jnp.int32)

            def ch_body(a, cnt):
                axv = ax_v[pl.ds(a * 16, 16)]
                ayv = ay_v[pl.ds(a * 16, 16)]
                dx = mxs - axv
                dy = mys - ayv
                d2 = dx * dx + dy * dy
                m = (d2 + 1e-6) <= 64.0
                sv = jnp.where(m, 1, 0)
                s = sv
                pbuf[pl.ds(16, 16)] = s
                s = s + pbuf[pl.ds(15, 16)]
                pbuf[pl.ds(16, 16)] = s
                s = s + pbuf[pl.ds(14, 16)]
                pbuf[pl.ds(16, 16)] = s
                s = s + pbuf[pl.ds(12, 16)]
                pbuf[pl.ds(16, 16)] = s
                pos = s + pbuf[pl.ds(8, 16)]
                n = pos[15]

                def do(cnt):
                    owi = zi16
                    odx = zf16
                    ody = zf16
                    for j in range(16):
                        pj = jnp.full((16,), pos[j] - 1, jnp.int32)
                        mj = jnp.full((16,), sv[j], jnp.int32)
                        eqi = jnp.where(lane == pj, mj, 0)
                        eqf = eqi.astype(jnp.float32)
                        owi = owi + eqi * jnp.full((16,), a * 16 + j,
                                                   jnp.int32)
                        odx = odx + eqf * jnp.full((16,), dx[j], jnp.float32)
                        ody = ody + eqf * jnp.full((16,), dy[j], jnp.float32)
                    hib[pl.ds(cnt, 16)] = hval
                    wib[pl.ds(cnt, 16)] = owi
                    rxb[pl.ds(cnt, 16)] = odx
                    ryb[pl.ds(cnt, 16)] = ody
                    return jnp.minimum(cnt + n, SEG)

                return lax.cond(n > 0, do, lambda c: c, cnt)

            return lax.fori_loop(0, N_ACT // 16, ch_body, cnt)

        cnt = lax.fori_loop(0, nrows, row_body, jnp.int32(0))

        cntv[...] = jnp.full((16,), cnt, jnp.int32)
        pltpu.sync_copy(hib.at[pl.ds(0, SEG)], hi_o.at[w])
        pltpu.sync_copy(wib.at[pl.ds(0, SEG)], wi_o.at[w])
        pltpu.sync_copy(rxb.at[pl.ds(0, SEG)], rx_o.at[w])
        pltpu.sync_copy(ryb.at[pl.ds(0, SEG)], ry_o.at[w])
        pltpu.sync_copy(cntv, cnt_o.at[w])

    return k(mxp, myp, ax, ay)


GCH = 128


def _sc_gather(qc, cc, hi2, wi2, counts):
    mesh = plsc.VectorSubcoreMesh(core_axis_name="c", subcore_axis_name="s")

    @functools.partial(
        pl.kernel,
        out_type=[
            jax.ShapeDtypeStruct((E_CAP, D), jnp.float32),
            jax.ShapeDtypeStruct((E_CAP, D), jnp.float32),
        ],
        mesh=mesh,
        scratch_types=[
            pltpu.VMEM((SEG,), jnp.int32),
            pltpu.VMEM((SEG,), jnp.int32),
            pltpu.VMEM((GCH, D), jnp.float32),
            pltpu.VMEM((GCH, D), jnp.float32),
            pltpu.VMEM((16,), jnp.int32),
        ],
    )
    def k(qc_h, cc_h, hi_h, wi_h, cnt_h, qg_o, cg_o,
          hi_v, wi_v, qbuf, cbuf, cntv):
        cid = lax.axis_index("c")
        sid = lax.axis_index("s")
        w = sid * 2 + cid
        pltpu.sync_copy(hi_h.at[w], hi_v)
        pltpu.sync_copy(wi_h.at[w], wi_v)
        pltpu.sync_copy(cnt_h.at[w], cntv)
        cnt = cntv[...][0]
        nch = (cnt + (GCH - 1)) // GCH

        def body(j, _):
            pltpu.sync_copy(qc_h.at[hi_v.at[pl.ds(j * GCH, GCH)]], qbuf)
            pltpu.sync_copy(cc_h.at[wi_v.at[pl.ds(j * GCH, GCH)]], cbuf)
            base = w * SEG + j * GCH
            pltpu.sync_copy(qbuf, qg_o.at[pl.ds(base, GCH)])
            pltpu.sync_copy(cbuf, cg_o.at[pl.ds(base, GCH)])
            return 0

        lax.fori_loop(0, nch, body, 0)

    return k(qc, cc, hi2, wi2, counts)


SCH = 64
RPS = 632
NMP = 16 * RPS


def _sc_scatter(c, hi3, counts, zrows):
    mesh = plsc.VectorSubcoreMesh(core_axis_name="c", subcore_axis_name="s")

    @functools.partial(
        pl.kernel,
        out_type=[
            jax.ShapeDtypeStruct((NMP, D), jnp.float32),
            jax.ShapeDtypeStruct((NMP, D), jnp.float32),
        ],
        mesh=mesh,
        scratch_types=[
            pltpu.VMEM_SHARED((NMP, D), jnp.float32),
            pltpu.VMEM((SEG // SCH, SCH), jnp.int32),
            pltpu.VMEM((SCH, D), jnp.float32),
            pltpu.VMEM((16,), jnp.int32),
        ],
    )
    def k(c_h, hi_h, cnt_h, z_h, p0_o, p1_o, acc, hiv, cbuf, cntv):
        cid = lax.axis_index("c")
        sid = lax.axis_index("s")
        w = sid * 2 + cid
        pltpu.sync_copy(z_h, acc.at[pl.ds(sid * RPS, RPS)])
        pltpu.sync_copy(hi_h.at[w], hiv)
        pltpu.sync_copy(cnt_h.at[w], cntv)
        cnt = cntv[...][0]
        nch = (cnt + (SCH - 1)) // SCH
        plsc.subcore_barrier()

        def body(j, _):
            pltpu.sync_copy(c_h.at[pl.ds(w * SEG + j * SCH, SCH)], cbuf)
            pltpu.sync_copy(cbuf, acc.at[hiv.at[j]], add=True)
            return 0

        lax.fori_loop(0, nch, body, 0)
        plsc.subcore_barrier()

        @pl.when(cid == 0)
        def _():
            pltpu.sync_copy(acc.at[pl.ds(sid * RPS, RPS)],
                            p0_o.at[pl.ds(sid * RPS, RPS)])

        @pl.when(cid == 1)
        def _():
            pltpu.sync_copy(acc.at[pl.ds(sid * RPS, RPS)],
                            p1_o.at[pl.ds(sid * RPS, RPS)])

    return k(c, hi3, counts, zrows)


def _att_params_prep(ap):
    return {
        'd0x': ap['dist0_W'][:, 0][None, :],
        'd0y': ap['dist0_W'][:, 1][None, :],
        'd0b': ap['dist0_b'][None, :],
        'w1T': ap['dist1_W'].T,
        'd1g': ap['dist1_g'][None, :],
        'd1b': ap['dist1_b'][None, :],
        'qT': ap['query_W'].T,
        'qg': ap['query_g'][None, :],
        'qb': ap['query_b'][None, :],
        'wdT': ap['ctx0_W'][:, 0:D].T,
        'wqT': ap['ctx0_W'][:, D:2 * D].T,
        'wcT': ap['ctx0_W'][:, 2 * D:3 * D].T,
        'c0g': ap['ctx0_g'][None, :],
        'c0b': ap['ctx0_b'][None, :],
        'c1T': ap['ctx1_W'].T,
        'agtT': ap['agt_W'].T,
        'ng': ap['norm_g'][None, :],
        'nb': ap['norm_b'][None, :],
        'linT': ap['lin_W'].T,
        'lg': ap['lin_g'][None, :],
        'lb': ap['lin_b'][None, :],
    }


def kernel(feat, turn, control, intersect, map_ctrs, actors, actor_ctrs, params):
    meta = jnp.concatenate([turn, control[:, None], intersect[:, None]], axis=1)
    wft = params['meta_W'][:, :D].T
    wmt = params['meta_W'][:, D:D + 4].T
    x = _k0(feat, meta, wft, wmt, params['meta_g'][None, :],
            params['meta_b'][None, :])

    mxp = jnp.zeros((MPAD,), jnp.float32).at[:N_MAP].set(map_ctrs[:, 0])
    myp = jnp.zeros((MPAD,), jnp.float32).at[:N_MAP].set(map_ctrs[:, 1])
    hi2, wi2, rx2, ry2, counts = _sc_edge_build(
        mxp, myp, actor_ctrs[:, 0], actor_ctrs[:, 1])
    hi = hi2.reshape(E_CAP)
    relx = rx2.reshape(E_CAP, 1)
    rely = ry2.reshape(E_CAP, 1)
    hi3 = hi.reshape(NW, SEG // SCH, SCH)
    zrows = jnp.zeros((RPS, D), jnp.float32)

    for l in range(2):
        p = _att_params_prep(params['att%d' % l])
        a0, qc, cc = _k1(x, actors, p['agtT'], p['qT'], p['qg'], p['qb'],
                         p['wqT'], p['wcT'])
        qg, cg = _sc_gather(qc, cc, hi2, wi2, counts)
        c = _k4(counts, relx, rely, qg, cg, p)
        p0f, p1f = _sc_scatter(c, hi3, counts, zrows)
        p0 = p0f[:N_MAP]
        p1 = p1f[:N_MAP]
        x = _k6(a0, p0, p1, x, p['linT'], p['ng'], p['nb'], p['lg'], p['lb'])
    return x
